# baseline (speedup 1.0000x reference)
import functools

import jax
import jax.numpy as jnp
from jax import lax
from jax.experimental import pallas as pl
from jax.experimental.pallas import tpu as pltpu
from jax.experimental.pallas import tpu_sc as plsc

N_NODES = 10000
N_EDGES = 320000
D_FEAT = 128
D_EDGE = 16
HID = 256


EDGE_BLK = 3200
EDGE_GRID = N_EDGES // EDGE_BLK


def _edge_mlp_body(eat_ref, w1_ref, b1_ref, w2_ref, b2_ref, we_ref, be_ref,
                   out_ref):
    blk_t = eat_ref[...]
    pol = jnp.reshape(blk_t[0, :], (EDGE_BLK, 1))
    pol = jnp.clip(pol, 0.0, 1.0) + 0.01
    h = jax.lax.dot_general(blk_t, w1_ref[...], (((0,), (0,)), ((), ())),
                            preferred_element_type=jnp.float32)
    h = jnp.maximum(h + b1_ref[...], 0.0)
    e = jnp.dot(h, w2_ref[...], preferred_element_type=jnp.float32)
    e = (e + b2_ref[...]) * pol
    o = jnp.dot(e, we_ref[...], preferred_element_type=jnp.float32)
    out_ref[...] = o + be_ref[...]


def _edge_mlp(edge_attr_t, W1p, b1, W2, b2, We, be, blk_off, n_edges):
    return pl.pallas_call(
        _edge_mlp_body,
        grid=(n_edges // EDGE_BLK,),
        in_specs=[
            pl.BlockSpec((D_EDGE, EDGE_BLK), lambda i, o=blk_off: (0, i + o)),
            pl.BlockSpec((D_EDGE, HID), lambda i: (0, 0)),
            pl.BlockSpec((1, HID), lambda i: (0, 0)),
            pl.BlockSpec((HID, HID), lambda i: (0, 0)),
            pl.BlockSpec((1, HID), lambda i: (0, 0)),
            pl.BlockSpec((HID, D_FEAT), lambda i: (0, 0)),
            pl.BlockSpec((1, D_FEAT), lambda i: (0, 0)),
        ],
        out_specs=pl.BlockSpec((EDGE_BLK, D_FEAT), lambda i: (i, 0)),
        out_shape=jax.ShapeDtypeStruct((n_edges, D_FEAT), jnp.float32),
    )(edge_attr_t, W1p, b1, W2, b2, We, be)


NC = 2
NS = 16
SPLIT_A = 96000
K = 64
NBUF = 4
XBUF = 2
UNROLL = 4
RPT = 624
REM = N_NODES - NS * RPT
NVEC = D_FEAT // 16


def _sc_body(edge_off, ept, chunks, ktail,
             x_hbm, ea_hbm, src_hbm, dst_hbm, out0, out1,
             idx_v, idx_t, xr_v, msg_v, agg_sh, lsem, gsem, ssem):
    EPT, CHUNKS, KTAIL = ept, chunks, ktail
    c = lax.axis_index("c")
    s = lax.axis_index("s")

    zero = jnp.zeros((16,), jnp.float32)

    def zfill_row(i, carry):
        for j in range(NVEC):
            msg_v[0, i, pl.ds(j * 16, 16)] = zero
        return carry

    lax.fori_loop(0, K, zfill_row, 0)

    my_rows = pl.multiple_of(s * RPT, 8)
    for q in range(RPT // K):
        pltpu.sync_copy(msg_v.at[0], agg_sh.at[pl.ds(my_rows + q * K, K)])
    zrem = RPT - (RPT // K) * K
    pltpu.sync_copy(msg_v.at[0, pl.ds(0, zrem)],
                    agg_sh.at[pl.ds(my_rows + (RPT // K) * K, zrem)])

    @pl.when(s == NS - 1)
    def _():
        pltpu.sync_copy(msg_v.at[0, pl.ds(0, REM)],
                        agg_sh.at[pl.ds(NS * RPT, REM)])

    plsc.subcore_barrier()

    edge_base = (c * NS + s) * EPT

    def issue_loads(ci, b):
        base = pl.multiple_of(edge_base + ci * K, 8)
        gbase = pl.multiple_of(edge_off + edge_base + ci * K, 8)
        pltpu.async_copy(src_hbm.at[pl.ds(gbase, K)], idx_v.at[b, 0],
                         lsem.at[b])
        pltpu.async_copy(dst_hbm.at[pl.ds(gbase, K)], idx_v.at[b, 1],
                         lsem.at[b])
        pltpu.async_copy(ea_hbm.at[pl.ds(base, K)], msg_v.at[b], lsem.at[b])

    def wait_loads(b):
        pltpu.make_async_copy(src_hbm.at[pl.ds(0, K)], idx_v.at[b, 0],
                              lsem.at[b]).wait()
        pltpu.make_async_copy(dst_hbm.at[pl.ds(0, K)], idx_v.at[b, 1],
                              lsem.at[b]).wait()
        pltpu.make_async_copy(ea_hbm.at[pl.ds(0, K)], msg_v.at[b],
                              lsem.at[b]).wait()

    def issue_gather(b, g):
        pltpu.async_copy(x_hbm.at[idx_v.at[b, 0]], xr_v.at[g], gsem.at[g])

    def wait_gather(b, g):
        pltpu.make_async_copy(x_hbm.at[idx_v.at[b, 0]], xr_v.at[g],
                              gsem.at[g]).wait()

    def issue_scatter(b):
        pltpu.async_copy(msg_v.at[b], agg_sh.at[idx_v.at[b, 1]], ssem.at[b],
                         add=True)

    def wait_scatter(b):
        pltpu.make_async_copy(msg_v.at[b], agg_sh.at[pl.ds(0, K)],
                              ssem.at[b]).wait()

    issue_loads(0, 0)
    issue_loads(1, 1)
    wait_loads(0)
    issue_gather(0, 0)

    def outer_body(g, carry):
        for b in range(UNROLL):
            ci = g * UNROLL + b
            b5 = b % NBUF
            b2 = b % XBUF

            @pl.when(ci + 1 < CHUNKS)
            def _():
                wait_loads((b + 1) % NBUF)
                issue_gather((b + 1) % NBUF, (b + 1) % XBUF)

            @pl.when((ci >= NBUF - 2) & (ci + 2 < CHUNKS))
            def _():
                wait_scatter((b + 2) % NBUF)

            @pl.when(ci + 2 < CHUNKS)
            def _():
                issue_loads(ci + 2, (b + 2) % NBUF)

            @pl.when(ci < CHUNKS)
            def _():
                wait_gather(b5, b2)

                def row_body(i, rcarry):
                    for j in range(NVEC):
                        sl = pl.ds(j * 16, 16)
                        msg_v[b5, i, sl] = jnp.maximum(
                            xr_v[b2, i, sl] + msg_v[b5, i, sl], 0.0)
                    return rcarry

                lax.fori_loop(0, K, row_body, 0)
                issue_scatter(b5)
        return carry

    lax.fori_loop(0, (CHUNKS + UNROLL - 1) // UNROLL, outer_body, 0)

    for q in range(NBUF):
        wait_scatter(q)

    if KTAIL > 0:
        tbase = pl.multiple_of(edge_base + CHUNKS * K, 8)
        gtbase = pl.multiple_of(edge_off + edge_base + CHUNKS * K, 8)
        pltpu.sync_copy(src_hbm.at[pl.ds(gtbase, KTAIL)], idx_t.at[0])
        pltpu.sync_copy(dst_hbm.at[pl.ds(gtbase, KTAIL)], idx_t.at[1])
        pltpu.sync_copy(ea_hbm.at[pl.ds(tbase, KTAIL)],
                        msg_v.at[0, pl.ds(0, KTAIL)])
        pltpu.async_copy(x_hbm.at[idx_t.at[0]], xr_v.at[0, pl.ds(0, KTAIL)],
                         gsem.at[0]).wait()

        def tail_row(i, rcarry):
            for j in range(NVEC):
                sl = pl.ds(j * 16, 16)
                msg_v[0, i, sl] = jnp.maximum(
                    xr_v[0, i, sl] + msg_v[0, i, sl], 0.0)
            return rcarry

        lax.fori_loop(0, KTAIL, tail_row, 0)
        pltpu.sync_copy(msg_v.at[0, pl.ds(0, KTAIL)], agg_sh.at[idx_t.at[1]],
                        add=True)
    plsc.subcore_barrier()

    @pl.when(c == 0)
    def _():
        pltpu.sync_copy(agg_sh.at[pl.ds(my_rows, RPT)],
                        out0.at[pl.ds(my_rows, RPT)])

        @pl.when(s == NS - 1)
        def _():
            pltpu.sync_copy(agg_sh.at[pl.ds(NS * RPT, REM)],
                            out0.at[pl.ds(NS * RPT, REM)])

    @pl.when(c == 1)
    def _():
        pltpu.sync_copy(agg_sh.at[pl.ds(my_rows, RPT)],
                        out1.at[pl.ds(my_rows, RPT)])

        @pl.when(s == NS - 1)
        def _():
            pltpu.sync_copy(agg_sh.at[pl.ds(NS * RPT, REM)],
                            out1.at[pl.ds(NS * RPT, REM)])


def _make_sc(edge_off, n_edges):
    ept = n_edges // (NC * NS)
    assert ept * NC * NS == n_edges and ept % 8 == 0
    chunks = ept // K
    ktail = ept - chunks * K
    assert ktail % 8 == 0
    return functools.partial(
        pl.kernel,
        out_type=(jax.ShapeDtypeStruct((N_NODES, D_FEAT), jnp.float32),
                  jax.ShapeDtypeStruct((N_NODES, D_FEAT), jnp.float32)),
        mesh=plsc.VectorSubcoreMesh(core_axis_name="c",
                                    subcore_axis_name="s"),
        scratch_types=[
            pltpu.VMEM((NBUF, 2, K), jnp.int32),
            pltpu.VMEM((2, max(ktail, 8)), jnp.int32),
            pltpu.VMEM((XBUF, K, D_FEAT), jnp.float32),
            pltpu.VMEM((NBUF, K, D_FEAT), jnp.float32),
            pltpu.VMEM_SHARED((N_NODES, D_FEAT), jnp.float32),
            pltpu.SemaphoreType.DMA((NBUF,)),
            pltpu.SemaphoreType.DMA((XBUF,)),
            pltpu.SemaphoreType.DMA((NBUF,)),
        ],
    )(functools.partial(_sc_body, edge_off, ept, chunks, ktail))


_sc_part0 = _make_sc(0, SPLIT_A)
_sc_part1 = _make_sc(SPLIT_A, N_EDGES - SPLIT_A)


NODE_BLK = 2000
NODE_GRID = N_NODES // NODE_BLK


def _node_mlp_body(x_ref, a0_ref, a1_ref, a2_ref, a3_ref, wn1_ref, bn1_ref,
                   g_ref, b_ref, wn2_ref, bn2_ref, out_ref):
    o = (x_ref[...] + (a0_ref[...] + a1_ref[...])
         + (a2_ref[...] + a3_ref[...]))
    h2 = jnp.dot(o, wn1_ref[...], preferred_element_type=jnp.float32)
    h2 = h2 + bn1_ref[...]
    mu = jnp.mean(h2, axis=-1, keepdims=True)
    var = jnp.mean((h2 - mu) ** 2, axis=-1, keepdims=True)
    h2 = (h2 - mu) / jnp.sqrt(var + 1e-5) * g_ref[...] + b_ref[...]
    h2 = jnp.maximum(h2, 0.0)
    o2 = jnp.dot(h2, wn2_ref[...], preferred_element_type=jnp.float32)
    out_ref[...] = o2 + bn2_ref[...]


def _node_mlp(x, a0, a1, a2, a3, Wn1, bn1, ln_g, ln_b, Wn2, bn2):
    full = lambda shape: pl.BlockSpec(shape, lambda i: (0, 0))
    blk = pl.BlockSpec((NODE_BLK, D_FEAT), lambda i: (i, 0))
    return pl.pallas_call(
        _node_mlp_body,
        grid=(NODE_GRID,),
        in_specs=[blk, blk, blk, blk, blk,
                  full((D_FEAT, D_FEAT)), full((1, D_FEAT)),
                  full((1, D_FEAT)), full((1, D_FEAT)),
                  full((D_FEAT, D_FEAT)), full((1, D_FEAT))],
        out_specs=blk,
        out_shape=jax.ShapeDtypeStruct((N_NODES, D_FEAT), jnp.float32),
    )(x, a0, a1, a2, a3, Wn1, bn1, ln_g, ln_b, Wn2, bn2)


def kernel(x, edge_index, edge_attr, W1, b1, W2, b2, We, be,
           Wn1, bn1, ln_g, ln_b, Wn2, bn2):
    W1p = jnp.concatenate([jnp.zeros((1, HID), W1.dtype), W1], axis=0)
    eat = edge_attr.T
    b1r, b2r, ber = b1[None, :], b2[None, :], be[None, :]
    ea0 = _edge_mlp(eat, W1p, b1r, W2, b2r, We, ber, 0, SPLIT_A)
    ea1 = _edge_mlp(eat, W1p, b1r, W2, b2r, We, ber, SPLIT_A // EDGE_BLK,
                    N_EDGES - SPLIT_A)
    src = edge_index[0].astype(jnp.int32)
    dst = edge_index[1].astype(jnp.int32)
    a00, a01 = _sc_part0(x, ea0, src, dst)
    a10, a11 = _sc_part1(x, ea1, src, dst)
    return _node_mlp(x, a00, a01, a10, a11, Wn1, bn1[None, :], ln_g[None, :],
                     ln_b[None, :], Wn2, bn2[None, :])


# speedup vs baseline: 1.0814x; 1.0814x over previous
import functools

import jax
import jax.numpy as jnp
from jax import lax
from jax.experimental import pallas as pl
from jax.experimental.pallas import tpu as pltpu
from jax.experimental.pallas import tpu_sc as plsc

N_NODES = 10000
N_EDGES = 320000
D_FEAT = 128
D_EDGE = 16
HID = 256


EDGE_BLK = 3200
EDGE_GRID = N_EDGES // EDGE_BLK


def _edge_mlp_body(eat_ref, w1_ref, b1_ref, w2_ref, b2_ref, we_ref, be_ref,
                   out_ref):
    blk_t = eat_ref[...]
    pol = jnp.reshape(blk_t[0, :], (EDGE_BLK, 1))
    pol = jnp.clip(pol, 0.0, 1.0) + 0.01
    h = jax.lax.dot_general(blk_t, w1_ref[...], (((0,), (0,)), ((), ())),
                            preferred_element_type=jnp.float32)
    h = jnp.maximum(h + b1_ref[...], 0.0)
    e = jnp.dot(h, w2_ref[...], preferred_element_type=jnp.float32)
    e = (e + b2_ref[...]) * pol
    o = jnp.dot(e, we_ref[...], preferred_element_type=jnp.float32)
    out_ref[...] = o + be_ref[...]


def _edge_mlp(edge_attr_t, W1p, b1, W2, b2, We, be, blk_off, n_edges):
    return pl.pallas_call(
        _edge_mlp_body,
        grid=(n_edges // EDGE_BLK,),
        in_specs=[
            pl.BlockSpec((D_EDGE, EDGE_BLK), lambda i, o=blk_off: (0, i + o)),
            pl.BlockSpec((D_EDGE, HID), lambda i: (0, 0)),
            pl.BlockSpec((1, HID), lambda i: (0, 0)),
            pl.BlockSpec((HID, HID), lambda i: (0, 0)),
            pl.BlockSpec((1, HID), lambda i: (0, 0)),
            pl.BlockSpec((HID, D_FEAT), lambda i: (0, 0)),
            pl.BlockSpec((1, D_FEAT), lambda i: (0, 0)),
        ],
        out_specs=pl.BlockSpec((EDGE_BLK, D_FEAT), lambda i: (i, 0)),
        out_shape=jax.ShapeDtypeStruct((n_edges, D_FEAT), jnp.float32),
    )(edge_attr_t, W1p, b1, W2, b2, We, be)


NC = 2
NS = 16
SPLIT_A = 160000
K = 64
NBUF = 4
XBUF = 2
UNROLL = 4
RPT = 624
REM = N_NODES - NS * RPT
NVEC = D_FEAT // 16


def _sc_body(edge_off, ept, chunks, ktail,
             x_hbm, ea_hbm, src_hbm, dst_hbm, out0, out1,
             idx_v, idx_t, xr_v, msg_v, agg_sh, lsem, gsem, ssem):
    EPT, CHUNKS, KTAIL = ept, chunks, ktail
    c = lax.axis_index("c")
    s = lax.axis_index("s")

    zero = jnp.zeros((16,), jnp.float32)

    def zfill_row(i, carry):
        for j in range(NVEC):
            msg_v[0, i, pl.ds(j * 16, 16)] = zero
        return carry

    lax.fori_loop(0, K, zfill_row, 0)

    my_rows = pl.multiple_of(s * RPT, 8)
    for q in range(RPT // K):
        pltpu.sync_copy(msg_v.at[0], agg_sh.at[pl.ds(my_rows + q * K, K)])
    zrem = RPT - (RPT // K) * K
    pltpu.sync_copy(msg_v.at[0, pl.ds(0, zrem)],
                    agg_sh.at[pl.ds(my_rows + (RPT // K) * K, zrem)])

    @pl.when(s == NS - 1)
    def _():
        pltpu.sync_copy(msg_v.at[0, pl.ds(0, REM)],
                        agg_sh.at[pl.ds(NS * RPT, REM)])

    plsc.subcore_barrier()

    edge_base = (c * NS + s) * EPT

    def issue_loads(ci, b):
        base = pl.multiple_of(edge_base + ci * K, 8)
        gbase = pl.multiple_of(edge_off + edge_base + ci * K, 8)
        pltpu.async_copy(src_hbm.at[pl.ds(gbase, K)], idx_v.at[b, 0],
                         lsem.at[b])
        pltpu.async_copy(dst_hbm.at[pl.ds(gbase, K)], idx_v.at[b, 1],
                         lsem.at[b])
        pltpu.async_copy(ea_hbm.at[pl.ds(base, K)], msg_v.at[b], lsem.at[b])

    def wait_loads(b):
        pltpu.make_async_copy(src_hbm.at[pl.ds(0, K)], idx_v.at[b, 0],
                              lsem.at[b]).wait()
        pltpu.make_async_copy(dst_hbm.at[pl.ds(0, K)], idx_v.at[b, 1],
                              lsem.at[b]).wait()
        pltpu.make_async_copy(ea_hbm.at[pl.ds(0, K)], msg_v.at[b],
                              lsem.at[b]).wait()

    def issue_gather(b, g):
        pltpu.async_copy(x_hbm.at[idx_v.at[b, 0]], xr_v.at[g], gsem.at[g])

    def wait_gather(b, g):
        pltpu.make_async_copy(x_hbm.at[idx_v.at[b, 0]], xr_v.at[g],
                              gsem.at[g]).wait()

    def issue_scatter(b):
        pltpu.async_copy(msg_v.at[b], agg_sh.at[idx_v.at[b, 1]], ssem.at[b],
                         add=True)

    def wait_scatter(b):
        pltpu.make_async_copy(msg_v.at[b], agg_sh.at[pl.ds(0, K)],
                              ssem.at[b]).wait()

    issue_loads(0, 0)
    issue_loads(1, 1)
    wait_loads(0)
    issue_gather(0, 0)

    def outer_body(g, carry):
        for b in range(UNROLL):
            ci = g * UNROLL + b
            b5 = b % NBUF
            b2 = b % XBUF

            @pl.when(ci + 1 < CHUNKS)
            def _():
                wait_loads((b + 1) % NBUF)
                issue_gather((b + 1) % NBUF, (b + 1) % XBUF)

            @pl.when((ci >= NBUF - 2) & (ci + 2 < CHUNKS))
            def _():
                wait_scatter((b + 2) % NBUF)

            @pl.when(ci + 2 < CHUNKS)
            def _():
                issue_loads(ci + 2, (b + 2) % NBUF)

            @pl.when(ci < CHUNKS)
            def _():
                wait_gather(b5, b2)

                def row_body(i, rcarry):
                    for j in range(NVEC):
                        sl = pl.ds(j * 16, 16)
                        msg_v[b5, i, sl] = jnp.maximum(
                            xr_v[b2, i, sl] + msg_v[b5, i, sl], 0.0)
                    return rcarry

                lax.fori_loop(0, K, row_body, 0)
                issue_scatter(b5)
        return carry

    lax.fori_loop(0, (CHUNKS + UNROLL - 1) // UNROLL, outer_body, 0)

    for q in range(NBUF):
        wait_scatter(q)

    if KTAIL > 0:
        tbase = pl.multiple_of(edge_base + CHUNKS * K, 8)
        gtbase = pl.multiple_of(edge_off + edge_base + CHUNKS * K, 8)
        pltpu.sync_copy(src_hbm.at[pl.ds(gtbase, KTAIL)], idx_t.at[0])
        pltpu.sync_copy(dst_hbm.at[pl.ds(gtbase, KTAIL)], idx_t.at[1])
        pltpu.sync_copy(ea_hbm.at[pl.ds(tbase, KTAIL)],
                        msg_v.at[0, pl.ds(0, KTAIL)])
        pltpu.async_copy(x_hbm.at[idx_t.at[0]], xr_v.at[0, pl.ds(0, KTAIL)],
                         gsem.at[0]).wait()

        def tail_row(i, rcarry):
            for j in range(NVEC):
                sl = pl.ds(j * 16, 16)
                msg_v[0, i, sl] = jnp.maximum(
                    xr_v[0, i, sl] + msg_v[0, i, sl], 0.0)
            return rcarry

        lax.fori_loop(0, KTAIL, tail_row, 0)
        pltpu.sync_copy(msg_v.at[0, pl.ds(0, KTAIL)], agg_sh.at[idx_t.at[1]],
                        add=True)
    plsc.subcore_barrier()

    @pl.when(c == 0)
    def _():
        pltpu.sync_copy(agg_sh.at[pl.ds(my_rows, RPT)],
                        out0.at[pl.ds(my_rows, RPT)])

        @pl.when(s == NS - 1)
        def _():
            pltpu.sync_copy(agg_sh.at[pl.ds(NS * RPT, REM)],
                            out0.at[pl.ds(NS * RPT, REM)])

    @pl.when(c == 1)
    def _():
        pltpu.sync_copy(agg_sh.at[pl.ds(my_rows, RPT)],
                        out1.at[pl.ds(my_rows, RPT)])

        @pl.when(s == NS - 1)
        def _():
            pltpu.sync_copy(agg_sh.at[pl.ds(NS * RPT, REM)],
                            out1.at[pl.ds(NS * RPT, REM)])


def _make_sc(edge_off, n_edges):
    ept = n_edges // (NC * NS)
    assert ept * NC * NS == n_edges and ept % 8 == 0
    chunks = ept // K
    ktail = ept - chunks * K
    assert ktail % 8 == 0
    return functools.partial(
        pl.kernel,
        out_type=(jax.ShapeDtypeStruct((N_NODES, D_FEAT), jnp.float32),
                  jax.ShapeDtypeStruct((N_NODES, D_FEAT), jnp.float32)),
        mesh=plsc.VectorSubcoreMesh(core_axis_name="c",
                                    subcore_axis_name="s"),
        scratch_types=[
            pltpu.VMEM((NBUF, 2, K), jnp.int32),
            pltpu.VMEM((2, max(ktail, 8)), jnp.int32),
            pltpu.VMEM((XBUF, K, D_FEAT), jnp.float32),
            pltpu.VMEM((NBUF, K, D_FEAT), jnp.float32),
            pltpu.VMEM_SHARED((N_NODES, D_FEAT), jnp.float32),
            pltpu.SemaphoreType.DMA((NBUF,)),
            pltpu.SemaphoreType.DMA((XBUF,)),
            pltpu.SemaphoreType.DMA((NBUF,)),
        ],
    )(functools.partial(_sc_body, edge_off, ept, chunks, ktail))


_sc_part0 = _make_sc(0, SPLIT_A)
_sc_part1 = _make_sc(SPLIT_A, N_EDGES - SPLIT_A)


NODE_BLK = 2000
NODE_GRID = N_NODES // NODE_BLK


def _node_mlp_body(x_ref, a0_ref, a1_ref, a2_ref, a3_ref, wn1_ref, bn1_ref,
                   g_ref, b_ref, wn2_ref, bn2_ref, out_ref):
    o = (x_ref[...] + (a0_ref[...] + a1_ref[...])
         + (a2_ref[...] + a3_ref[...]))
    h2 = jnp.dot(o, wn1_ref[...], preferred_element_type=jnp.float32)
    h2 = h2 + bn1_ref[...]
    mu = jnp.mean(h2, axis=-1, keepdims=True)
    var = jnp.mean((h2 - mu) ** 2, axis=-1, keepdims=True)
    h2 = (h2 - mu) / jnp.sqrt(var + 1e-5) * g_ref[...] + b_ref[...]
    h2 = jnp.maximum(h2, 0.0)
    o2 = jnp.dot(h2, wn2_ref[...], preferred_element_type=jnp.float32)
    out_ref[...] = o2 + bn2_ref[...]


def _node_mlp(x, a0, a1, a2, a3, Wn1, bn1, ln_g, ln_b, Wn2, bn2):
    full = lambda shape: pl.BlockSpec(shape, lambda i: (0, 0))
    blk = pl.BlockSpec((NODE_BLK, D_FEAT), lambda i: (i, 0))
    return pl.pallas_call(
        _node_mlp_body,
        grid=(NODE_GRID,),
        in_specs=[blk, blk, blk, blk, blk,
                  full((D_FEAT, D_FEAT)), full((1, D_FEAT)),
                  full((1, D_FEAT)), full((1, D_FEAT)),
                  full((D_FEAT, D_FEAT)), full((1, D_FEAT))],
        out_specs=blk,
        out_shape=jax.ShapeDtypeStruct((N_NODES, D_FEAT), jnp.float32),
    )(x, a0, a1, a2, a3, Wn1, bn1, ln_g, ln_b, Wn2, bn2)


def kernel(x, edge_index, edge_attr, W1, b1, W2, b2, We, be,
           Wn1, bn1, ln_g, ln_b, Wn2, bn2):
    W1p = jnp.concatenate([jnp.zeros((1, HID), W1.dtype), W1], axis=0)
    eat = edge_attr.T
    b1r, b2r, ber = b1[None, :], b2[None, :], be[None, :]
    ea0 = _edge_mlp(eat, W1p, b1r, W2, b2r, We, ber, 0, SPLIT_A)
    ea1 = _edge_mlp(eat, W1p, b1r, W2, b2r, We, ber, SPLIT_A // EDGE_BLK,
                    N_EDGES - SPLIT_A)
    src = edge_index[0].astype(jnp.int32)
    dst = edge_index[1].astype(jnp.int32)
    a00, a01 = _sc_part0(x, ea0, src, dst)
    a10, a11 = _sc_part1(x, ea1, src, dst)
    return _node_mlp(x, a00, a01, a10, a11, Wn1, bn1[None, :], ln_g[None, :],
                     ln_b[None, :], Wn2, bn2[None, :])
